# baseline (speedup 1.0000x reference)
import functools

import jax
import jax.numpy as jnp
from jax import lax
from jax.experimental import pallas as pl
from jax.experimental.pallas import tpu as pltpu
from jax.experimental.pallas import tpu_sc as plsc

B = 4096
D = 64

_NC = 2
_NS = 16
_NW = _NC * _NS
_BPW = B // _NW


@functools.cache
def _make_sc_gather():
    mesh = plsc.VectorSubcoreMesh(core_axis_name="c", subcore_axis_name="s")

    @functools.partial(
        pl.kernel,
        mesh=mesh,
        out_type=[
            jax.ShapeDtypeStruct((B, 2 * D), jnp.float32),
            jax.ShapeDtypeStruct((B, 2 * D), jnp.float32),
        ],
        scratch_types=[
            pltpu.VMEM((_BPW,), jnp.int32),
            pltpu.VMEM((_BPW,), jnp.int32),
            pltpu.VMEM((_BPW, 2 * D), jnp.float32),
            pltpu.VMEM((_BPW, 2 * D), jnp.float32),
            pltpu.SemaphoreType.DMA,
            pltpu.SemaphoreType.DMA,
        ],
    )
    def _sc_gather(uidx_hbm, iidx_hbm, utab_hbm, itab_hbm, uout_hbm, iout_hbm,
                   uidx_v, iidx_v, urows_v, irows_v, usem, isem):
        wid = lax.axis_index("s") * _NC + lax.axis_index("c")
        base = wid * _BPW
        pltpu.sync_copy(uidx_hbm.at[pl.ds(base, _BPW)], uidx_v)
        pltpu.sync_copy(iidx_hbm.at[pl.ds(base, _BPW)], iidx_v)
        ucp = pltpu.async_copy(utab_hbm.at[uidx_v], urows_v, usem)
        icp = pltpu.async_copy(itab_hbm.at[iidx_v], irows_v, isem)
        ucp.wait()
        icp.wait()
        pltpu.sync_copy(urows_v, uout_hbm.at[pl.ds(base, _BPW)])
        pltpu.sync_copy(irows_v, iout_hbm.at[pl.ds(base, _BPW)])

    return _sc_gather


_BM = 512
_BN = 1024


def _mm_body(uh_ref, ih_ref, up_ref, ip_ref, o_ref):
    u = jnp.where(uh_ref[...] != 0, up_ref[:, D:], up_ref[:, :D])
    i = jnp.where(ih_ref[...] != 0, ip_ref[:, D:], ip_ref[:, :D])
    o_ref[...] = lax.dot_general(
        u, i, (((1,), (1,)), ((), ())), preferred_element_type=jnp.float32)


_matmul = pl.pallas_call(
    _mm_body,
    grid=(B // _BM, B // _BN),
    in_specs=[
        pl.BlockSpec((_BM, 1), lambda i, j: (i, 0)),
        pl.BlockSpec((_BN, 1), lambda i, j: (j, 0)),
        pl.BlockSpec((_BM, 2 * D), lambda i, j: (i, 0)),
        pl.BlockSpec((_BN, 2 * D), lambda i, j: (j, 0)),
    ],
    out_specs=pl.BlockSpec((_BM, _BN), lambda i, j: (i, j)),
    out_shape=jax.ShapeDtypeStruct((B, B), jnp.float32),
)


@jax.jit
def kernel(user_indices, item_indices, user_table, item_table):
    uidx = user_indices.astype(jnp.int32)
    iidx = item_indices.astype(jnp.int32)
    utab2 = user_table.reshape(user_table.shape[0] // 2, 2 * D)
    itab2 = item_table.reshape(item_table.shape[0] // 2, 2 * D)
    upairs, ipairs = _make_sc_gather()(
        uidx >> 1, iidx >> 1, utab2, itab2)
    uh = (uidx & 1).reshape(B, 1)
    ih = (iidx & 1).reshape(B, 1)
    return _matmul(uh, ih, upairs, ipairs)


# speedup vs baseline: 1.3761x; 1.3761x over previous
import functools

import jax
import jax.numpy as jnp
from jax import lax
from jax.experimental import pallas as pl
from jax.experimental.pallas import tpu as pltpu

B = 4096
D = 64
_NSEM = 8
_CHUNK = B // _NSEM


def _gather_body(uidx_ref, iidx_ref, utab_ref, itab_ref, uout_ref, iout_ref,
                 sems):
    def issue(k, _):
        for j in range(_NSEM):
            row = k * _NSEM + j
            pltpu.make_async_copy(
                utab_ref.at[pl.ds(uidx_ref[row], 1)],
                uout_ref.at[pl.ds(row, 1)],
                sems.at[j],
            ).start()
            pltpu.make_async_copy(
                itab_ref.at[pl.ds(iidx_ref[row], 1)],
                iout_ref.at[pl.ds(row, 1)],
                sems.at[j],
            ).start()
        return 0
    lax.fori_loop(0, B // _NSEM, issue, 0, unroll=True)

    for j in range(_NSEM):
        pltpu.make_async_copy(
            utab_ref.at[pl.ds(0, 2 * _CHUNK)],
            uout_ref.at[pl.ds(0, 2 * _CHUNK)],
            sems.at[j],
        ).wait()


_gather = pl.pallas_call(
    _gather_body,
    in_specs=[
        pl.BlockSpec(memory_space=pltpu.SMEM),
        pl.BlockSpec(memory_space=pltpu.SMEM),
        pl.BlockSpec(memory_space=pl.ANY),
        pl.BlockSpec(memory_space=pl.ANY),
    ],
    out_specs=[
        pl.BlockSpec(memory_space=pl.ANY),
        pl.BlockSpec(memory_space=pl.ANY),
    ],
    out_shape=[
        jax.ShapeDtypeStruct((B, D), jnp.float32),
        jax.ShapeDtypeStruct((B, D), jnp.float32),
    ],
    scratch_shapes=[pltpu.SemaphoreType.DMA((_NSEM,))],
)


_BM = 512
_BN = 1024


def _mm_body(u_ref, i_ref, o_ref):
    o_ref[...] = lax.dot_general(
        u_ref[...], i_ref[...],
        (((1,), (1,)), ((), ())),
        preferred_element_type=jnp.float32,
    )


_matmul = pl.pallas_call(
    _mm_body,
    grid=(B // _BM, B // _BN),
    in_specs=[
        pl.BlockSpec((_BM, D), lambda i, j: (i, 0)),
        pl.BlockSpec((_BN, D), lambda i, j: (j, 0)),
    ],
    out_specs=pl.BlockSpec((_BM, _BN), lambda i, j: (i, j)),
    out_shape=jax.ShapeDtypeStruct((B, B), jnp.float32),
)


@jax.jit
def kernel(user_indices, item_indices, user_table, item_table):
    user_embs, item_embs = _gather(
        user_indices.astype(jnp.int32), item_indices.astype(jnp.int32),
        user_table, item_table)
    return _matmul(user_embs, item_embs)


# speedup vs baseline: 1.5442x; 1.1222x over previous
import functools

import jax
import jax.numpy as jnp
from jax import lax
from jax.experimental import pallas as pl
from jax.experimental.pallas import tpu as pltpu

B = 4096
D = 64
_NSEM = 8
_CHUNK = B // _NSEM


def _gather_body(uidx_ref, iidx_ref, utab_ref, itab_ref, uout_ref, iout_ref,
                 uvmem, ivmem, sems, osem):
    def issue(k, _):
        for j in range(_NSEM):
            row = k * _NSEM + j
            pltpu.make_async_copy(
                utab_ref.at[pl.ds(uidx_ref[row], 1)],
                uvmem.at[pl.ds(row, 1)],
                sems.at[j],
            ).start()
            pltpu.make_async_copy(
                itab_ref.at[pl.ds(iidx_ref[row], 1)],
                ivmem.at[pl.ds(row, 1)],
                sems.at[j],
            ).start()
        return 0
    lax.fori_loop(0, B // _NSEM, issue, 0, unroll=True)

    for j in range(_NSEM):
        pltpu.make_async_copy(
            utab_ref.at[pl.ds(0, 2 * _CHUNK)],
            uvmem.at[pl.ds(0, 2 * _CHUNK)],
            sems.at[j],
        ).wait()

    ucp = pltpu.make_async_copy(uvmem, uout_ref, osem)
    icp = pltpu.make_async_copy(ivmem, iout_ref, osem)
    ucp.start()
    icp.start()
    ucp.wait()
    icp.wait()


_gather = pl.pallas_call(
    _gather_body,
    in_specs=[
        pl.BlockSpec(memory_space=pltpu.SMEM),
        pl.BlockSpec(memory_space=pltpu.SMEM),
        pl.BlockSpec(memory_space=pl.ANY),
        pl.BlockSpec(memory_space=pl.ANY),
    ],
    out_specs=[
        pl.BlockSpec(memory_space=pl.ANY),
        pl.BlockSpec(memory_space=pl.ANY),
    ],
    out_shape=[
        jax.ShapeDtypeStruct((B, D), jnp.float32),
        jax.ShapeDtypeStruct((B, D), jnp.float32),
    ],
    scratch_shapes=[
        pltpu.VMEM((B, D), jnp.float32),
        pltpu.VMEM((B, D), jnp.float32),
        pltpu.SemaphoreType.DMA((_NSEM,)),
        pltpu.SemaphoreType.DMA,
    ],
)


_BM = 512
_BN = 1024


def _mm_body(u_ref, i_ref, o_ref):
    o_ref[...] = lax.dot_general(
        u_ref[...], i_ref[...],
        (((1,), (1,)), ((), ())),
        preferred_element_type=jnp.float32,
    )


_matmul = pl.pallas_call(
    _mm_body,
    grid=(B // _BM, B // _BN),
    in_specs=[
        pl.BlockSpec((_BM, D), lambda i, j: (i, 0)),
        pl.BlockSpec((_BN, D), lambda i, j: (j, 0)),
    ],
    out_specs=pl.BlockSpec((_BM, _BN), lambda i, j: (i, j)),
    out_shape=jax.ShapeDtypeStruct((B, B), jnp.float32),
)


@jax.jit
def kernel(user_indices, item_indices, user_table, item_table):
    user_embs, item_embs = _gather(
        user_indices.astype(jnp.int32), item_indices.astype(jnp.int32),
        user_table, item_table)
    return _matmul(user_embs, item_embs)
